# Initial kernel scaffold; baseline (speedup 1.0000x reference)
import jax
import jax.numpy as jnp
from jax.experimental import pallas as pl


def kernel(ged, y):
    raise NotImplementedError("write your pallas kernel here")


# speedup vs baseline: 36.7012x; 36.7012x over previous
import functools

import jax
import jax.numpy as jnp
from jax import lax
from jax.experimental import pallas as pl
from jax.experimental.pallas import tpu as pltpu
from jax.experimental.pallas import tpu_sc as plsc

_N_TRAIN = 128
_K = 16
_L = 16
_W = 256


def _merge16(ak, ap, bk, bp, do_sort):
    rbk = jnp.flip(bk, 0)
    rbp = jnp.flip(bp, 0)
    m = ak <= rbk
    nk = jnp.where(m, ak, rbk)
    np_ = jnp.where(m, ap, rbp)
    if do_sort:
        nk, np_ = plsc.sort_key_val(nk, np_)
    return nk, np_


def kernel(ged, y):
    n_train, n_query = ged.shape
    info = plsc.get_sparse_core_info()
    nc, ns = info.num_cores, info.num_subcores
    nw = nc * ns
    q_per_w = n_query // nw
    n_tiles = q_per_w // _W
    n_leaves = _N_TRAIN // _L

    mesh = plsc.VectorSubcoreMesh(core_axis_name="c", subcore_axis_name="s")

    @functools.partial(
        pl.kernel,
        mesh=mesh,
        out_type=jax.ShapeDtypeStruct((n_query,), jnp.float32),
        scratch_types=[
            pltpu.VMEM((_N_TRAIN, _W), jnp.float32),
            pltpu.VMEM((_W,), jnp.float32),
            pltpu.VMEM((_N_TRAIN,), jnp.float32),
        ],
        compiler_params=pltpu.CompilerParams(
            use_tc_tiling_on_sc=False, needs_layout_passes=False
        ),
    )
    def sc_knn(ged_hbm, y_hbm, out_hbm, tile_v, out_v, y_v):
        wid = lax.axis_index("s") * nc + lax.axis_index("c")
        pltpu.sync_copy(y_hbm, y_v)
        iota = lax.iota(jnp.int32, _L)
        lane0 = iota == 0
        row_idx = [iota + _L * j for j in range(n_leaves)]
        y_leaf = [y_v[pl.ds(_L * j, _L)] for j in range(n_leaves)]

        def tile_body(t, carry):
            base = wid * q_per_w + t * _W
            pltpu.sync_copy(ged_hbm.at[:, pl.ds(base, _W)], tile_v)

            def q_body(q, carry2):
                col = jnp.full((_L,), q, jnp.int32)
                vs = [
                    plsc.load_gather(tile_v, [row_idx[j], col])
                    for j in range(n_leaves)
                ]
                sq = vs[0] * vs[0]
                for j in range(1, n_leaves):
                    sq = sq + vs[j] * vs[j]
                s_tot = jnp.sum(sq)
                s_vec = jnp.maximum(jnp.full((_L,), s_tot, jnp.float32), 1e-30)
                bits = plsc.bitcast(s_vec, jnp.int32)
                r = plsc.bitcast(0x5F3759DF - (bits >> 1), jnp.float32)
                for _ in range(3):
                    r = r * (1.5 - 0.5 * s_vec * r * r)
                norm = jnp.maximum(s_vec * r, 1e-12)

                kv = [
                    plsc.sort_key_val(vs[j], y_leaf[j])
                    for j in range(n_leaves)
                ]
                m0 = _merge16(*kv[0], *kv[1], True)
                m1 = _merge16(*kv[2], *kv[3], True)
                m2 = _merge16(*kv[4], *kv[5], True)
                m3 = _merge16(*kv[6], *kv[7], True)
                p0 = _merge16(*m0, *m1, True)
                p1 = _merge16(*m2, *m3, True)
                fk, fp = _merge16(*p0, *p1, False)

                sim = norm / (fk + norm)
                den = jnp.sum(sim)
                num = jnp.sum(sim * fp)
                res = jnp.full((_L,), num, jnp.float32) / jnp.full(
                    (_L,), den, jnp.float32
                )
                plsc.store_scatter(out_v, [col], res, mask=lane0)
                return carry2

            lax.fori_loop(0, _W, q_body, 0)
            pltpu.sync_copy(out_v, out_hbm.at[pl.ds(base, _W)])
            return carry

        lax.fori_loop(0, n_tiles, tile_body, 0)

    return sc_knn(ged, y)


# speedup vs baseline: 51.6723x; 1.4079x over previous
import functools

import jax
import jax.numpy as jnp
from jax import lax
from jax.experimental import pallas as pl
from jax.experimental.pallas import tpu as pltpu
from jax.experimental.pallas import tpu_sc as plsc

_N_TRAIN = 128
_K = 16
_L = 16
_W = 256


def _merge16(ak, ap, bk, bp, do_sort):
    rbk = jnp.flip(bk, 0)
    rbp = jnp.flip(bp, 0)
    m = ak <= rbk
    nk = jnp.where(m, ak, rbk)
    np_ = jnp.where(m, ap, rbp)
    if do_sort:
        nk, np_ = plsc.sort_key_val(nk, np_)
    return nk, np_


def kernel(ged, y):
    n_train, n_query = ged.shape
    info = plsc.get_sparse_core_info()
    nc, ns = info.num_cores, info.num_subcores
    nw = nc * ns
    q_per_w = n_query // nw
    n_tiles = q_per_w // _W
    n_leaves = _N_TRAIN // _L

    mesh = plsc.VectorSubcoreMesh(core_axis_name="c", subcore_axis_name="s")

    @functools.partial(
        pl.kernel,
        mesh=mesh,
        out_type=jax.ShapeDtypeStruct((n_query,), jnp.float32),
        scratch_types=[
            pltpu.VMEM((_N_TRAIN, _W), jnp.float32),
            pltpu.VMEM((_W,), jnp.float32),
            pltpu.VMEM((_N_TRAIN,), jnp.float32),
        ],
        compiler_params=pltpu.CompilerParams(
            use_tc_tiling_on_sc=False, needs_layout_passes=False
        ),
    )
    def sc_knn(ged_hbm, y_hbm, out_hbm, tile_v, out_v, y_v):
        wid = lax.axis_index("s") * nc + lax.axis_index("c")
        pltpu.sync_copy(y_hbm, y_v)
        iota = lax.iota(jnp.int32, _L)
        lane0 = iota == 0
        row_idx = [iota + _L * j for j in range(n_leaves)]
        y_leaf = [y_v[pl.ds(_L * j, _L)] for j in range(n_leaves)]

        def tile_body(t, carry):
            base = wid * q_per_w + t * _W
            pltpu.sync_copy(ged_hbm.at[:, pl.ds(base, _W)], tile_v)

            @plsc.parallel_loop(0, _W, 1, unroll=4)
            def q_body(q):
                col = jnp.full((_L,), q, jnp.int32)
                vs = [
                    plsc.load_gather(tile_v, [row_idx[j], col])
                    for j in range(n_leaves)
                ]
                sq = vs[0] * vs[0]
                for j in range(1, n_leaves):
                    sq = sq + vs[j] * vs[j]
                s_tot = jnp.sum(sq)
                s_vec = jnp.maximum(jnp.full((_L,), s_tot, jnp.float32), 1e-30)
                bits = plsc.bitcast(s_vec, jnp.int32)
                r = plsc.bitcast(0x5F3759DF - (bits >> 1), jnp.float32)
                for _ in range(3):
                    r = r * (1.5 - 0.5 * s_vec * r * r)
                norm = jnp.maximum(s_vec * r, 1e-12)

                kv = [
                    plsc.sort_key_val(vs[j], y_leaf[j])
                    for j in range(n_leaves)
                ]
                m0 = _merge16(*kv[0], *kv[1], True)
                m1 = _merge16(*kv[2], *kv[3], True)
                m2 = _merge16(*kv[4], *kv[5], True)
                m3 = _merge16(*kv[6], *kv[7], True)
                p0 = _merge16(*m0, *m1, True)
                p1 = _merge16(*m2, *m3, True)
                fk, fp = _merge16(*p0, *p1, False)

                sim = norm / (fk + norm)
                den = jnp.sum(sim)
                num = jnp.sum(sim * fp)
                res = jnp.full((_L,), num, jnp.float32) / jnp.full(
                    (_L,), den, jnp.float32
                )
                plsc.store_scatter(out_v, [col], res, mask=lane0)

            pltpu.sync_copy(out_v, out_hbm.at[pl.ds(base, _W)])
            return carry

        lax.fori_loop(0, n_tiles, tile_body, 0)

    return sc_knn(ged, y)
